# baseline (speedup 1.0000x reference)
import functools
import math

import jax
import jax.numpy as jnp
from jax import lax
from jax.experimental import pallas as pl
from jax.experimental.pallas import tpu as pltpu
from jax.experimental.pallas import tpu_sc as plsc

H = 16
P = 4


def _proj_body(x_ref, kv_ref, wq_ref, bq_ref, wk_ref, bk_ref, wv_ref, bv_ref,
               woff_ref, boff_ref, wattn_ref, battn_ref,
               qh_ref, k_ref, v_ref, off_ref, lg_ref):
    hd = qh_ref.shape[-1]
    x = x_ref[0]
    kv = kv_ref[0]
    q = jnp.dot(x, wq_ref[...], preferred_element_type=jnp.float32) + bq_ref[...]
    k = jnp.dot(kv, wk_ref[...], preferred_element_type=jnp.float32) + bk_ref[...]
    v = jnp.dot(kv, wv_ref[...], preferred_element_type=jnp.float32) + bv_ref[...]
    off = jnp.dot(x, woff_ref[...], preferred_element_type=jnp.float32) + boff_ref[...]
    lg = jnp.dot(x, wattn_ref[...], preferred_element_type=jnp.float32) + battn_ref[...]
    k_ref[0] = k
    v_ref[0] = v
    off_ref[0] = off
    lg_ref[0] = lg
    scale = 1.0 / math.sqrt(hd)
    for h in range(H):
        qh_ref[0, h] = q[:, h * hd:(h + 1) * hd] * scale


def _projections(q_in, kv_in, Wq, bq, Wk, bk, Wv, bv, Woff, boff, Wattn, battn):
    B, L, D = q_in.shape
    HD = D // H
    LB = 256
    grid = (B, L // LB)
    full = lambda shape: pl.BlockSpec(shape, lambda b, i: (0,) * len(shape))
    row_spec = pl.BlockSpec((1, LB, D), lambda b, i: (b, i, 0))
    out_shapes = (
        jax.ShapeDtypeStruct((B, H, L, HD), jnp.float32),
        jax.ShapeDtypeStruct((B, L, D), jnp.float32),
        jax.ShapeDtypeStruct((B, L, D), jnp.float32),
        jax.ShapeDtypeStruct((B, L, H * P), jnp.float32),
        jax.ShapeDtypeStruct((B, L, H * P), jnp.float32),
    )
    return pl.pallas_call(
        _proj_body,
        grid=grid,
        in_specs=[
            row_spec, row_spec,
            full((D, D)), full((D,)), full((D, D)), full((D,)),
            full((D, D)), full((D,)), full((D, H * P)), full((H * P,)),
            full((D, H * P)), full((H * P,)),
        ],
        out_specs=(
            pl.BlockSpec((1, H, LB, HD), lambda b, i: (b, 0, i, 0)),
            row_spec,
            row_spec,
            pl.BlockSpec((1, LB, H * P), lambda b, i: (b, i, 0)),
            pl.BlockSpec((1, LB, H * P), lambda b, i: (b, i, 0)),
        ),
        out_shape=out_shapes,
    )(q_in, kv_in, Wq, bq, Wk, bk, Wv, bv, Woff, boff, Wattn, battn)


def _build_body(k_ref, kn_ref, v_ref, vn_ref, kvp_ref):
    hd = kvp_ref.shape[-1] // 4
    k = k_ref[0]
    kn = kn_ref[0]
    v = v_ref[0]
    vn = vn_ref[0]
    for h in range(H):
        sl = slice(h * hd, (h + 1) * hd)
        kh = k[:, sl]
        vh = v[:, sl]
        kh1 = jnp.concatenate([kh[1:], kn[:1, sl]], axis=0)
        vh1 = jnp.concatenate([vh[1:], vn[:1, sl]], axis=0)
        kvp_ref[0, h] = jnp.concatenate([kh, kh1, vh, vh1], axis=1)


def _build_pairs(k, v):
    B, L, D = k.shape
    HD = D // H
    LB = 256
    nblk = L // LB
    grid = (B, nblk)
    cur = pl.BlockSpec((1, LB, D), lambda b, i: (b, i, 0))
    nxt = pl.BlockSpec((1, LB, D),
                       lambda b, i: (b, jnp.minimum(i + 1, nblk - 1), 0))
    return pl.pallas_call(
        _build_body,
        grid=grid,
        in_specs=[cur, nxt, cur, nxt],
        out_specs=pl.BlockSpec((1, H, LB, 4 * HD), lambda b, i: (b, 0, i, 0)),
        out_shape=jax.ShapeDtypeStruct((B, H, L, 4 * HD), jnp.float32),
    )(k, k, v, v)


def _attn_body(q_ref, g_ref, meta_ref, ctx_ref):
    hd = q_ref.shape[-1]
    lb = q_ref.shape[-2]
    q = q_ref[0, 0]
    meta = meta_ref[0, 0]
    parts = []
    for p in range(P):
        gp = g_ref[p, 0, 0]
        w0 = meta[:, p:p + 1]
        w1 = meta[:, P + p:P + p + 1]
        ks = gp[:, :hd] * w0 + gp[:, hd:2 * hd] * w1
        parts.append(ks * q)
    prod = jnp.concatenate(parts, axis=1)
    seg = jax.lax.broadcasted_iota(jnp.int32, (P * hd, P), 0) // hd
    col = jax.lax.broadcasted_iota(jnp.int32, (P * hd, P), 1)
    S = (seg == col).astype(jnp.float32)
    scores = jnp.dot(prod, S, preferred_element_type=jnp.float32)
    scores = scores + meta[:, 2 * P:3 * P]
    m = jnp.max(scores, axis=1, keepdims=True)
    e = jnp.exp(scores - m)
    z = jnp.sum(e, axis=1, keepdims=True)
    wgt = e / z
    ctx = jnp.zeros((lb, hd), jnp.float32)
    for p in range(P):
        gp = g_ref[p, 0, 0]
        c0 = wgt[:, p:p + 1] * meta[:, p:p + 1]
        c1 = wgt[:, p:p + 1] * meta[:, P + p:P + p + 1]
        ctx = ctx + c0 * gp[:, 2 * hd:3 * hd] + c1 * gp[:, 3 * hd:4 * hd]
    ctx_ref[0, 0] = ctx


def _attention(qh, gath5, meta, h0, hg):
    B, Hh, L, HD = qh.shape
    LB = 512
    grid = (B, hg, L // LB)
    return pl.pallas_call(
        _attn_body,
        grid=grid,
        in_specs=[
            pl.BlockSpec((1, 1, LB, HD), lambda b, h, i: (b, h0 + h, i, 0)),
            pl.BlockSpec((P, 1, 1, LB, 4 * HD), lambda b, h, i: (0, b, h, i, 0)),
            pl.BlockSpec((1, 1, LB, 64), lambda b, h, i: (b, h0 + h, i, 0)),
        ],
        out_specs=pl.BlockSpec((1, 1, LB, HD), lambda b, h, i: (b, h, i, 0)),
        out_shape=jax.ShapeDtypeStruct((B, hg, L, HD), jnp.float32),
    )(qh, gath5, meta)


def _outproj_body(c0_ref, c1_ref, c2_ref, c3_ref, w_ref, b_ref, o_ref):
    hg = c0_ref.shape[1]
    x = jnp.concatenate(
        [c_ref[0, h] for c_ref in (c0_ref, c1_ref, c2_ref, c3_ref)
         for h in range(hg)], axis=1)
    o_ref[0] = jnp.dot(x, w_ref[...], preferred_element_type=jnp.float32) + b_ref[...]


def _outproj(ctxs, Wout, bout):
    B, hg, L, HD = ctxs[0].shape
    D = H * HD
    LB = 256
    grid = (B, L // LB)
    cspec = pl.BlockSpec((1, hg, LB, HD), lambda b, i: (b, 0, i, 0))
    return pl.pallas_call(
        _outproj_body,
        grid=grid,
        in_specs=[
            cspec, cspec, cspec, cspec,
            pl.BlockSpec((D, D), lambda b, i: (0, 0)),
            pl.BlockSpec((D,), lambda b, i: (0,)),
        ],
        out_specs=pl.BlockSpec((1, LB, D), lambda b, i: (b, i, 0)),
        out_shape=jax.ShapeDtypeStruct((B, L, D), jnp.float32),
    )(*ctxs, Wout, bout)


def _sc_gather(table, gidx):
    R = gidx.shape[0]
    W = table.shape[1]
    NC = 2
    NS = 16
    NW = NC * NS
    r_per_w = R // NW
    CH = 64
    NBUF = 4
    n_chunks = r_per_w // CH
    mesh = plsc.VectorSubcoreMesh(core_axis_name="c", subcore_axis_name="s")

    @functools.partial(
        pl.kernel,
        out_type=jax.ShapeDtypeStruct((R, W), jnp.float32),
        mesh=mesh,
        scratch_types=[
            pltpu.VMEM((NBUF, CH), jnp.int32),
            pltpu.VMEM((NBUF, CH, W), jnp.float32),
            pltpu.SemaphoreType.DMA((NBUF,)),
            pltpu.SemaphoreType.DMA((NBUF,)),
        ],
    )
    def gather_kernel(tab_hbm, idx_hbm, out_hbm, idx_v, rows_v, sem_g, sem_o):
        wid = lax.axis_index("s") * NC + lax.axis_index("c")
        base = wid * r_per_w

        def fill(c, b):
            pltpu.sync_copy(idx_hbm.at[pl.ds(base + c * CH, CH)], idx_v.at[b])
            pltpu.async_copy(tab_hbm.at[idx_v.at[b]], rows_v.at[b],
                             sem_g.at[b])

        def wait_fill(b):
            pltpu.make_async_copy(tab_hbm.at[idx_v.at[b]], rows_v.at[b],
                                  sem_g.at[b]).wait()

        def drain(c, b):
            pltpu.async_copy(rows_v.at[b], out_hbm.at[pl.ds(base + c * CH, CH)],
                             sem_o.at[b])

        def wait_drain(c, b):
            pltpu.make_async_copy(rows_v.at[b],
                                  out_hbm.at[pl.ds(base + c * CH, CH)],
                                  sem_o.at[b]).wait()

        for b in range(NBUF):
            fill(b, b)

        @pl.loop(0, n_chunks - NBUF, step=NBUF)
        def _(c):
            for b in range(NBUF):
                wait_fill(b)
                drain(c + b, b)
            for b in range(NBUF):
                wait_drain(c + b, b)
                fill(c + NBUF + b, b)

        for b in range(NBUF):
            wait_fill(b)
            drain(n_chunks - NBUF + b, b)
        for b in range(NBUF):
            wait_drain(n_chunks - NBUF + b, b)

    return gather_kernel(table, gidx)


@jax.jit
def kernel(q_in, kv_in, Wq, bq, Wk, bk, Wv, bv, Woff, boff, Wattn, battn,
           Wout, bout):
    B, L, D = q_in.shape
    HD = D // H

    qh, k, v, off, lg = _projections(q_in, kv_in, Wq, bq, Wk, bk, Wv, bv,
                                     Woff, boff, Wattn, battn)
    kvp = _build_pairs(k, v)

    offT = off.reshape(B, L, H, P).transpose(0, 2, 1, 3)
    lgT = lg.reshape(B, L, H, P).transpose(0, 2, 1, 3)
    basef = jnp.arange(L, dtype=jnp.float32).reshape(1, 1, L, 1)
    idxf = jnp.clip(basef + offT, 0.0, float(L - 1))
    base = jnp.clip(jnp.floor(idxf), 0.0, float(L - 2))
    w1 = idxf - base
    w0 = 1.0 - w1
    meta = jnp.concatenate(
        [w0, w1, lgT, jnp.zeros((B, H, L, 64 - 3 * P), jnp.float32)], axis=-1)
    bh = jnp.arange(B * H, dtype=jnp.int32).reshape(B, H, 1, 1)
    rowid = (bh * L + base.astype(jnp.int32)).transpose(3, 0, 1, 2)

    G = 4
    hg = H // G
    kvp_flat = kvp.reshape(B * H * L, 4 * HD)
    gaths = [
        _sc_gather(kvp_flat, rowid[:, :, g * hg:(g + 1) * hg, :].reshape(-1))
        for g in range(G)
    ]
    ctxs = [
        _attention(qh, gaths[g].reshape(P, B, hg, L, 4 * HD), meta,
                   g * hg, hg)
        for g in range(G)
    ]
    return _outproj(ctxs, Wout, bout)


# speedup vs baseline: 1.0878x; 1.0878x over previous
import functools
import math

import jax
import jax.numpy as jnp
from jax import lax
from jax.experimental import pallas as pl
from jax.experimental.pallas import tpu as pltpu
from jax.experimental.pallas import tpu_sc as plsc

H = 16
P = 4


def _proj_body(x_ref, kv_ref, wq_ref, bq_ref, wk_ref, bk_ref, wv_ref, bv_ref,
               woff_ref, boff_ref, wattn_ref, battn_ref,
               qh_ref, k_ref, v_ref, off_ref, lg_ref):
    hd = qh_ref.shape[-1]
    x = x_ref[0]
    kv = kv_ref[0]
    q = jnp.dot(x, wq_ref[...], preferred_element_type=jnp.float32) + bq_ref[...]
    k = jnp.dot(kv, wk_ref[...], preferred_element_type=jnp.float32) + bk_ref[...]
    v = jnp.dot(kv, wv_ref[...], preferred_element_type=jnp.float32) + bv_ref[...]
    off = jnp.dot(x, woff_ref[...], preferred_element_type=jnp.float32) + boff_ref[...]
    lg = jnp.dot(x, wattn_ref[...], preferred_element_type=jnp.float32) + battn_ref[...]
    k_ref[0] = k
    v_ref[0] = v
    off_ref[0] = off
    lg_ref[0] = lg
    scale = 1.0 / math.sqrt(hd)
    for h in range(H):
        qh_ref[0, h] = q[:, h * hd:(h + 1) * hd] * scale


def _projections(q_in, kv_in, Wq, bq, Wk, bk, Wv, bv, Woff, boff, Wattn, battn):
    B, L, D = q_in.shape
    HD = D // H
    LB = 256
    grid = (B, L // LB)
    full = lambda shape: pl.BlockSpec(shape, lambda b, i: (0,) * len(shape))
    row_spec = pl.BlockSpec((1, LB, D), lambda b, i: (b, i, 0))
    out_shapes = (
        jax.ShapeDtypeStruct((B, H, L, HD), jnp.float32),
        jax.ShapeDtypeStruct((B, L, D), jnp.float32),
        jax.ShapeDtypeStruct((B, L, D), jnp.float32),
        jax.ShapeDtypeStruct((B, L, H * P), jnp.float32),
        jax.ShapeDtypeStruct((B, L, H * P), jnp.float32),
    )
    return pl.pallas_call(
        _proj_body,
        grid=grid,
        in_specs=[
            row_spec, row_spec,
            full((D, D)), full((D,)), full((D, D)), full((D,)),
            full((D, D)), full((D,)), full((D, H * P)), full((H * P,)),
            full((D, H * P)), full((H * P,)),
        ],
        out_specs=(
            pl.BlockSpec((1, H, LB, HD), lambda b, i: (b, 0, i, 0)),
            row_spec,
            row_spec,
            pl.BlockSpec((1, LB, H * P), lambda b, i: (b, i, 0)),
            pl.BlockSpec((1, LB, H * P), lambda b, i: (b, i, 0)),
        ),
        out_shape=out_shapes,
    )(q_in, kv_in, Wq, bq, Wk, bk, Wv, bv, Woff, boff, Wattn, battn)


def _build_body(k_ref, kn_ref, v_ref, vn_ref, kvp_ref):
    hd = kvp_ref.shape[-1] // 4
    k = k_ref[0]
    kn = kn_ref[0]
    v = v_ref[0]
    vn = vn_ref[0]
    for h in range(H):
        sl = slice(h * hd, (h + 1) * hd)
        kh = k[:, sl]
        vh = v[:, sl]
        kh1 = jnp.concatenate([kh[1:], kn[:1, sl]], axis=0)
        vh1 = jnp.concatenate([vh[1:], vn[:1, sl]], axis=0)
        kvp_ref[0, h] = jnp.concatenate([kh, kh1, vh, vh1], axis=1)


def _build_pairs(k, v):
    B, L, D = k.shape
    HD = D // H
    LB = 256
    nblk = L // LB
    grid = (B, nblk)
    cur = pl.BlockSpec((1, LB, D), lambda b, i: (b, i, 0))
    nxt = pl.BlockSpec((1, LB, D),
                       lambda b, i: (b, jnp.minimum(i + 1, nblk - 1), 0))
    return pl.pallas_call(
        _build_body,
        grid=grid,
        in_specs=[cur, nxt, cur, nxt],
        out_specs=pl.BlockSpec((1, H, LB, 4 * HD), lambda b, i: (b, 0, i, 0)),
        out_shape=jax.ShapeDtypeStruct((B, H, L, 4 * HD), jnp.float32),
    )(k, k, v, v)


def _mm(a, b):
    return jnp.dot(a, b, preferred_element_type=jnp.float32)


def _attn_body(q_ref, g_ref, meta_ref, ctx_ref):
    hd = q_ref.shape[-1]
    q = q_ref[0, 0]
    meta = meta_ref[0, 0]
    segcol = jax.lax.broadcasted_iota(jnp.int32, (2 * P, 2 * P * hd), 1)
    segrow = jax.lax.broadcasted_iota(jnp.int32, (2 * P, 2 * P * hd), 0)
    src = (segcol // hd) % 2 * P + segcol // (2 * hd)
    WC = (segrow == src).astype(jnp.float32)
    drow = jax.lax.broadcasted_iota(jnp.int32, (hd, 2 * P * hd), 0)
    dcol = jax.lax.broadcasted_iota(jnp.int32, (hd, 2 * P * hd), 1)
    QE = (drow == dcol % hd).astype(jnp.float32)
    prow = jax.lax.broadcasted_iota(jnp.int32, (2 * P * hd, P), 0)
    pcol = jax.lax.broadcasted_iota(jnp.int32, (2 * P * hd, P), 1)
    S4 = (prow // (2 * hd) == pcol).astype(jnp.float32)

    kcat = jnp.concatenate([g_ref[p, 0, 0][:, :2 * hd] for p in range(P)],
                           axis=1)
    wbig = _mm(meta[:, :2 * P], WC)
    qbig = _mm(q, QE)
    prod = kcat * wbig * qbig
    scores = _mm(prod, S4) + meta[:, 2 * P:3 * P]
    m = jnp.max(scores, axis=1, keepdims=True)
    e = jnp.exp(scores - m)
    z = jnp.sum(e, axis=1, keepdims=True)
    wgt = e / z

    vcat = jnp.concatenate([g_ref[p, 0, 0][:, 2 * hd:] for p in range(P)],
                           axis=1)
    ccat = jnp.concatenate([wgt * meta[:, :P], wgt * meta[:, P:2 * P]],
                           axis=1)
    cbig = _mm(ccat, WC)
    ctxfull = vcat * cbig
    ctx = ctxfull[:, :hd]
    for s in range(1, 2 * P):
        ctx = ctx + ctxfull[:, s * hd:(s + 1) * hd]
    ctx_ref[0, 0] = ctx


def _attention(qh, gath5, meta, h0, hg):
    B, Hh, L, HD = qh.shape
    LB = 1024
    grid = (B, hg, L // LB)
    return pl.pallas_call(
        _attn_body,
        grid=grid,
        in_specs=[
            pl.BlockSpec((1, 1, LB, HD), lambda b, h, i: (b, h0 + h, i, 0)),
            pl.BlockSpec((P, 1, 1, LB, 4 * HD), lambda b, h, i: (0, b, h, i, 0)),
            pl.BlockSpec((1, 1, LB, 64), lambda b, h, i: (b, h0 + h, i, 0)),
        ],
        out_specs=pl.BlockSpec((1, 1, LB, HD), lambda b, h, i: (b, h, i, 0)),
        out_shape=jax.ShapeDtypeStruct((B, hg, L, HD), jnp.float32),
    )(qh, gath5, meta)


def _outproj_body(c0_ref, c1_ref, c2_ref, c3_ref, w_ref, b_ref, o_ref):
    hg = c0_ref.shape[1]
    x = jnp.concatenate(
        [c_ref[0, h] for c_ref in (c0_ref, c1_ref, c2_ref, c3_ref)
         for h in range(hg)], axis=1)
    o_ref[0] = jnp.dot(x, w_ref[...], preferred_element_type=jnp.float32) + b_ref[...]


def _outproj(ctxs, Wout, bout):
    B, hg, L, HD = ctxs[0].shape
    D = H * HD
    LB = 256
    grid = (B, L // LB)
    cspec = pl.BlockSpec((1, hg, LB, HD), lambda b, i: (b, 0, i, 0))
    return pl.pallas_call(
        _outproj_body,
        grid=grid,
        in_specs=[
            cspec, cspec, cspec, cspec,
            pl.BlockSpec((D, D), lambda b, i: (0, 0)),
            pl.BlockSpec((D,), lambda b, i: (0,)),
        ],
        out_specs=pl.BlockSpec((1, LB, D), lambda b, i: (b, i, 0)),
        out_shape=jax.ShapeDtypeStruct((B, L, D), jnp.float32),
    )(*ctxs, Wout, bout)


def _sc_gather(table, gidx):
    R = gidx.shape[0]
    W = table.shape[1]
    NC = 2
    NS = 16
    NW = NC * NS
    r_per_w = R // NW
    CH = 64
    NBUF = 4
    n_chunks = r_per_w // CH
    mesh = plsc.VectorSubcoreMesh(core_axis_name="c", subcore_axis_name="s")

    @functools.partial(
        pl.kernel,
        out_type=jax.ShapeDtypeStruct((R, W), jnp.float32),
        mesh=mesh,
        scratch_types=[
            pltpu.VMEM((NBUF, CH), jnp.int32),
            pltpu.VMEM((NBUF, CH, W), jnp.float32),
            pltpu.SemaphoreType.DMA((NBUF,)),
            pltpu.SemaphoreType.DMA((NBUF,)),
        ],
    )
    def gather_kernel(tab_hbm, idx_hbm, out_hbm, idx_v, rows_v, sem_g, sem_o):
        wid = lax.axis_index("s") * NC + lax.axis_index("c")
        base = wid * r_per_w

        def fill(c, b):
            pltpu.sync_copy(idx_hbm.at[pl.ds(base + c * CH, CH)], idx_v.at[b])
            pltpu.async_copy(tab_hbm.at[idx_v.at[b]], rows_v.at[b],
                             sem_g.at[b])

        def wait_fill(b):
            pltpu.make_async_copy(tab_hbm.at[idx_v.at[b]], rows_v.at[b],
                                  sem_g.at[b]).wait()

        def drain(c, b):
            pltpu.async_copy(rows_v.at[b], out_hbm.at[pl.ds(base + c * CH, CH)],
                             sem_o.at[b])

        def wait_drain(c, b):
            pltpu.make_async_copy(rows_v.at[b],
                                  out_hbm.at[pl.ds(base + c * CH, CH)],
                                  sem_o.at[b]).wait()

        for b in range(NBUF):
            fill(b, b)

        @pl.loop(0, n_chunks - NBUF, step=NBUF)
        def _(c):
            for b in range(NBUF):
                wait_fill(b)
                drain(c + b, b)
            for b in range(NBUF):
                wait_drain(c + b, b)
                fill(c + NBUF + b, b)

        for b in range(NBUF):
            wait_fill(b)
            drain(n_chunks - NBUF + b, b)
        for b in range(NBUF):
            wait_drain(n_chunks - NBUF + b, b)

    return gather_kernel(table, gidx)


@jax.jit
def kernel(q_in, kv_in, Wq, bq, Wk, bk, Wv, bv, Woff, boff, Wattn, battn,
           Wout, bout):
    B, L, D = q_in.shape
    HD = D // H

    qh, k, v, off, lg = _projections(q_in, kv_in, Wq, bq, Wk, bk, Wv, bv,
                                     Woff, boff, Wattn, battn)
    kvp = _build_pairs(k, v)

    offT = off.reshape(B, L, H, P).transpose(0, 2, 1, 3)
    lgT = lg.reshape(B, L, H, P).transpose(0, 2, 1, 3)
    basef = jnp.arange(L, dtype=jnp.float32).reshape(1, 1, L, 1)
    idxf = jnp.clip(basef + offT, 0.0, float(L - 1))
    base = jnp.clip(jnp.floor(idxf), 0.0, float(L - 2))
    w1 = idxf - base
    w0 = 1.0 - w1
    meta = jnp.concatenate(
        [w0, w1, lgT, jnp.zeros((B, H, L, 64 - 3 * P), jnp.float32)], axis=-1)
    bh = jnp.arange(B * H, dtype=jnp.int32).reshape(B, H, 1, 1)
    rowid = (bh * L + base.astype(jnp.int32)).transpose(3, 0, 1, 2)

    G = 4
    hg = H // G
    kvp_flat = kvp.reshape(B * H * L, 4 * HD)
    gaths = [
        _sc_gather(kvp_flat, rowid[:, :, g * hg:(g + 1) * hg, :].reshape(-1))
        for g in range(G)
    ]
    ctxs = [
        _attention(qh, gaths[g].reshape(P, B, hg, L, 4 * HD), meta,
                   g * hg, hg)
        for g in range(G)
    ]
    return _outproj(ctxs, Wout, bout)


# speedup vs baseline: 1.1377x; 1.0458x over previous
import functools
import math

import jax
import jax.numpy as jnp
from jax import lax
from jax.experimental import pallas as pl
from jax.experimental.pallas import tpu as pltpu
from jax.experimental.pallas import tpu_sc as plsc

H = 16
P = 4


def _proj_body(x_ref, kv_ref, wq_ref, bq_ref, wk_ref, bk_ref, wv_ref, bv_ref,
               woff_ref, boff_ref, wattn_ref, battn_ref,
               qh_ref, k_ref, v_ref, off_ref, lg_ref):
    hd = qh_ref.shape[-1]
    x = x_ref[0]
    kv = kv_ref[0]
    q = jnp.dot(x, wq_ref[...], preferred_element_type=jnp.float32) + bq_ref[...]
    k = jnp.dot(kv, wk_ref[...], preferred_element_type=jnp.float32) + bk_ref[...]
    v = jnp.dot(kv, wv_ref[...], preferred_element_type=jnp.float32) + bv_ref[...]
    off = jnp.dot(x, woff_ref[...], preferred_element_type=jnp.float32) + boff_ref[...]
    lg = jnp.dot(x, wattn_ref[...], preferred_element_type=jnp.float32) + battn_ref[...]
    k_ref[0] = k
    v_ref[0] = v
    off_ref[0] = off
    lg_ref[0] = lg
    scale = 1.0 / math.sqrt(hd)
    for h in range(H):
        qh_ref[0, h] = q[:, h * hd:(h + 1) * hd] * scale


def _projections(q_in, kv_in, Wq, bq, Wk, bk, Wv, bv, Woff, boff, Wattn, battn):
    B, L, D = q_in.shape
    HD = D // H
    LB = 256
    grid = (B, L // LB)
    full = lambda shape: pl.BlockSpec(shape, lambda b, i: (0,) * len(shape))
    row_spec = pl.BlockSpec((1, LB, D), lambda b, i: (b, i, 0))
    out_shapes = (
        jax.ShapeDtypeStruct((B, H, L, HD), jnp.float32),
        jax.ShapeDtypeStruct((B, L, D), jnp.float32),
        jax.ShapeDtypeStruct((B, L, D), jnp.float32),
        jax.ShapeDtypeStruct((B, L, H * P), jnp.float32),
        jax.ShapeDtypeStruct((B, L, H * P), jnp.float32),
    )
    return pl.pallas_call(
        _proj_body,
        grid=grid,
        in_specs=[
            row_spec, row_spec,
            full((D, D)), full((D,)), full((D, D)), full((D,)),
            full((D, D)), full((D,)), full((D, H * P)), full((H * P,)),
            full((D, H * P)), full((H * P,)),
        ],
        out_specs=(
            pl.BlockSpec((1, H, LB, HD), lambda b, i: (b, 0, i, 0)),
            row_spec,
            row_spec,
            pl.BlockSpec((1, LB, H * P), lambda b, i: (b, i, 0)),
            pl.BlockSpec((1, LB, H * P), lambda b, i: (b, i, 0)),
        ),
        out_shape=out_shapes,
    )(q_in, kv_in, Wq, bq, Wk, bk, Wv, bv, Woff, boff, Wattn, battn)


def _build_body(k_ref, kn_ref, v_ref, vn_ref, kvp_ref):
    hd = kvp_ref.shape[-1] // 4
    k = k_ref[0]
    kn = kn_ref[0]
    v = v_ref[0]
    vn = vn_ref[0]
    for h in range(H):
        sl = slice(h * hd, (h + 1) * hd)
        kh = k[:, sl]
        vh = v[:, sl]
        kh1 = jnp.concatenate([kh[1:], kn[:1, sl]], axis=0)
        vh1 = jnp.concatenate([vh[1:], vn[:1, sl]], axis=0)
        kvp_ref[0, h] = jnp.concatenate([kh, kh1, vh, vh1], axis=1)


def _build_pairs(k, v):
    B, L, D = k.shape
    HD = D // H
    LB = 256
    nblk = L // LB
    grid = (B, nblk)
    cur = pl.BlockSpec((1, LB, D), lambda b, i: (b, i, 0))
    nxt = pl.BlockSpec((1, LB, D),
                       lambda b, i: (b, jnp.minimum(i + 1, nblk - 1), 0))
    return pl.pallas_call(
        _build_body,
        grid=grid,
        in_specs=[cur, nxt, cur, nxt],
        out_specs=pl.BlockSpec((1, H, LB, 4 * HD), lambda b, i: (b, 0, i, 0)),
        out_shape=jax.ShapeDtypeStruct((B, H, L, 4 * HD), jnp.float32),
    )(k, k, v, v)


def _mm(a, b):
    return jnp.dot(a, b, preferred_element_type=jnp.float32)


def _attn_body(q_ref, g_ref, meta_ref, ctx_ref):
    hd = q_ref.shape[-1]
    q = q_ref[0, 0]
    meta = meta_ref[0, 0]
    segcol = jax.lax.broadcasted_iota(jnp.int32, (2 * P, 2 * P * hd), 1)
    segrow = jax.lax.broadcasted_iota(jnp.int32, (2 * P, 2 * P * hd), 0)
    src = (segcol // hd) % 2 * P + segcol // (2 * hd)
    WC = (segrow == src).astype(jnp.float32)
    drow = jax.lax.broadcasted_iota(jnp.int32, (hd, 2 * P * hd), 0)
    dcol = jax.lax.broadcasted_iota(jnp.int32, (hd, 2 * P * hd), 1)
    QE = (drow == dcol % hd).astype(jnp.float32)
    prow = jax.lax.broadcasted_iota(jnp.int32, (2 * P * hd, P), 0)
    pcol = jax.lax.broadcasted_iota(jnp.int32, (2 * P * hd, P), 1)
    S4 = (prow // (2 * hd) == pcol).astype(jnp.float32)

    kcat = jnp.concatenate([g_ref[p, 0, 0][:, :2 * hd] for p in range(P)],
                           axis=1)
    wbig = _mm(meta[:, :2 * P], WC)
    qbig = _mm(q, QE)
    prod = kcat * wbig * qbig
    scores = jnp.dot(prod.astype(jnp.bfloat16), S4.astype(jnp.bfloat16),
                     preferred_element_type=jnp.float32)
    scores = scores + meta[:, 2 * P:3 * P]
    m = jnp.max(scores, axis=1, keepdims=True)
    e = jnp.exp(scores - m)
    z = jnp.sum(e, axis=1, keepdims=True)
    wgt = e / z

    vcat = jnp.concatenate([g_ref[p, 0, 0][:, 2 * hd:] for p in range(P)],
                           axis=1)
    ccat = jnp.concatenate([wgt * meta[:, :P], wgt * meta[:, P:2 * P]],
                           axis=1)
    cbig = _mm(ccat, WC)
    ctxfull = vcat * cbig
    ctx = ctxfull[:, :hd]
    for s in range(1, 2 * P):
        ctx = ctx + ctxfull[:, s * hd:(s + 1) * hd]
    ctx_ref[0, 0] = ctx


def _attention(qh, gath5, meta, h0, hg):
    B, Hh, L, HD = qh.shape
    LB = 1024
    grid = (B, hg, L // LB)
    return pl.pallas_call(
        _attn_body,
        grid=grid,
        in_specs=[
            pl.BlockSpec((1, 1, LB, HD), lambda b, h, i: (b, h0 + h, i, 0)),
            pl.BlockSpec((P, 1, 1, LB, 4 * HD), lambda b, h, i: (0, b, h, i, 0)),
            pl.BlockSpec((1, 1, LB, 64), lambda b, h, i: (b, h0 + h, i, 0)),
        ],
        out_specs=pl.BlockSpec((1, 1, LB, HD), lambda b, h, i: (b, h, i, 0)),
        out_shape=jax.ShapeDtypeStruct((B, hg, L, HD), jnp.float32),
    )(qh, gath5, meta)


def _outproj_body(c0_ref, c1_ref, c2_ref, c3_ref, w_ref, b_ref, o_ref):
    hg = c0_ref.shape[1]
    x = jnp.concatenate(
        [c_ref[0, h] for c_ref in (c0_ref, c1_ref, c2_ref, c3_ref)
         for h in range(hg)], axis=1)
    o_ref[0] = jnp.dot(x, w_ref[...], preferred_element_type=jnp.float32) + b_ref[...]


def _outproj(ctxs, Wout, bout):
    B, hg, L, HD = ctxs[0].shape
    D = H * HD
    LB = 256
    grid = (B, L // LB)
    cspec = pl.BlockSpec((1, hg, LB, HD), lambda b, i: (b, 0, i, 0))
    return pl.pallas_call(
        _outproj_body,
        grid=grid,
        in_specs=[
            cspec, cspec, cspec, cspec,
            pl.BlockSpec((D, D), lambda b, i: (0, 0)),
            pl.BlockSpec((D,), lambda b, i: (0,)),
        ],
        out_specs=pl.BlockSpec((1, LB, D), lambda b, i: (b, i, 0)),
        out_shape=jax.ShapeDtypeStruct((B, L, D), jnp.float32),
    )(*ctxs, Wout, bout)


def _sc_gather(table, gidx):
    R = gidx.shape[0]
    W = table.shape[1]
    NC = 2
    NS = 16
    NW = NC * NS
    r_per_w = R // NW
    CH = 64
    NBUF = 4
    n_chunks = r_per_w // CH
    mesh = plsc.VectorSubcoreMesh(core_axis_name="c", subcore_axis_name="s")

    @functools.partial(
        pl.kernel,
        out_type=jax.ShapeDtypeStruct((R, W), jnp.float32),
        mesh=mesh,
        scratch_types=[
            pltpu.VMEM((NBUF, CH), jnp.int32),
            pltpu.VMEM((NBUF, CH, W), jnp.float32),
            pltpu.SemaphoreType.DMA((NBUF,)),
            pltpu.SemaphoreType.DMA((NBUF,)),
        ],
    )
    def gather_kernel(tab_hbm, idx_hbm, out_hbm, idx_v, rows_v, sem_g, sem_o):
        wid = lax.axis_index("s") * NC + lax.axis_index("c")
        base = wid * r_per_w

        def fill(c, b):
            pltpu.sync_copy(idx_hbm.at[pl.ds(base + c * CH, CH)], idx_v.at[b])
            pltpu.async_copy(tab_hbm.at[idx_v.at[b]], rows_v.at[b],
                             sem_g.at[b])

        def wait_fill(b):
            pltpu.make_async_copy(tab_hbm.at[idx_v.at[b]], rows_v.at[b],
                                  sem_g.at[b]).wait()

        def drain(c, b):
            pltpu.async_copy(rows_v.at[b], out_hbm.at[pl.ds(base + c * CH, CH)],
                             sem_o.at[b])

        def wait_drain(c, b):
            pltpu.make_async_copy(rows_v.at[b],
                                  out_hbm.at[pl.ds(base + c * CH, CH)],
                                  sem_o.at[b]).wait()

        for b in range(NBUF):
            fill(b, b)

        @pl.loop(0, n_chunks - NBUF, step=NBUF)
        def _(c):
            for b in range(NBUF):
                wait_fill(b)
                drain(c + b, b)
            for b in range(NBUF):
                wait_drain(c + b, b)
                fill(c + NBUF + b, b)

        for b in range(NBUF):
            wait_fill(b)
            drain(n_chunks - NBUF + b, b)
        for b in range(NBUF):
            wait_drain(n_chunks - NBUF + b, b)

    return gather_kernel(table, gidx)


@jax.jit
def kernel(q_in, kv_in, Wq, bq, Wk, bk, Wv, bv, Woff, boff, Wattn, battn,
           Wout, bout):
    B, L, D = q_in.shape
    HD = D // H

    qh, k, v, off, lg = _projections(q_in, kv_in, Wq, bq, Wk, bk, Wv, bv,
                                     Woff, boff, Wattn, battn)
    kvp = _build_pairs(k, v)

    offT = off.reshape(B, L, H, P).transpose(0, 2, 1, 3)
    lgT = lg.reshape(B, L, H, P).transpose(0, 2, 1, 3)
    basef = jnp.arange(L, dtype=jnp.float32).reshape(1, 1, L, 1)
    idxf = jnp.clip(basef + offT, 0.0, float(L - 1))
    base = jnp.clip(jnp.floor(idxf), 0.0, float(L - 2))
    w1 = idxf - base
    w0 = 1.0 - w1
    meta = jnp.concatenate(
        [w0, w1, lgT, jnp.zeros((B, H, L, 64 - 3 * P), jnp.float32)], axis=-1)
    bh = jnp.arange(B * H, dtype=jnp.int32).reshape(B, H, 1, 1)
    rowid = (bh * L + base.astype(jnp.int32)).transpose(3, 0, 1, 2)

    G = 4
    hg = H // G
    kvp_flat = kvp.reshape(B * H * L, 4 * HD)
    gaths = [
        _sc_gather(kvp_flat, rowid[:, :, g * hg:(g + 1) * hg, :].reshape(-1))
        for g in range(G)
    ]
    ctxs = [
        _attention(qh, gaths[g].reshape(P, B, hg, L, 4 * HD), meta,
                   g * hg, hg)
        for g in range(G)
    ]
    return _outproj(ctxs, Wout, bout)


# speedup vs baseline: 1.1411x; 1.0030x over previous
import functools
import math

import jax
import jax.numpy as jnp
from jax import lax
from jax.experimental import pallas as pl
from jax.experimental.pallas import tpu as pltpu
from jax.experimental.pallas import tpu_sc as plsc

H = 16
P = 4


def _proj_body(x_ref, kv_ref, wq_ref, bq_ref, wk_ref, bk_ref, wv_ref, bv_ref,
               woff_ref, boff_ref, wattn_ref, battn_ref,
               qh_ref, k_ref, v_ref, off_ref, lg_ref):
    hd = qh_ref.shape[-1]
    x = x_ref[0]
    kv = kv_ref[0]
    q = jnp.dot(x, wq_ref[...], preferred_element_type=jnp.float32) + bq_ref[...]
    k = jnp.dot(kv, wk_ref[...], preferred_element_type=jnp.float32) + bk_ref[...]
    v = jnp.dot(kv, wv_ref[...], preferred_element_type=jnp.float32) + bv_ref[...]
    off = jnp.dot(x, woff_ref[...], preferred_element_type=jnp.float32) + boff_ref[...]
    lg = jnp.dot(x, wattn_ref[...], preferred_element_type=jnp.float32) + battn_ref[...]
    k_ref[0] = k
    v_ref[0] = v
    off_ref[0] = off
    lg_ref[0] = lg
    scale = 1.0 / math.sqrt(hd)
    for h in range(H):
        qh_ref[0, h] = q[:, h * hd:(h + 1) * hd] * scale


def _projections(q_in, kv_in, Wq, bq, Wk, bk, Wv, bv, Woff, boff, Wattn, battn):
    B, L, D = q_in.shape
    HD = D // H
    LB = 256
    grid = (B, L // LB)
    full = lambda shape: pl.BlockSpec(shape, lambda b, i: (0,) * len(shape))
    row_spec = pl.BlockSpec((1, LB, D), lambda b, i: (b, i, 0))
    out_shapes = (
        jax.ShapeDtypeStruct((B, H, L, HD), jnp.float32),
        jax.ShapeDtypeStruct((B, L, D), jnp.float32),
        jax.ShapeDtypeStruct((B, L, D), jnp.float32),
        jax.ShapeDtypeStruct((B, L, H * P), jnp.float32),
        jax.ShapeDtypeStruct((B, L, H * P), jnp.float32),
    )
    return pl.pallas_call(
        _proj_body,
        grid=grid,
        in_specs=[
            row_spec, row_spec,
            full((D, D)), full((D,)), full((D, D)), full((D,)),
            full((D, D)), full((D,)), full((D, H * P)), full((H * P,)),
            full((D, H * P)), full((H * P,)),
        ],
        out_specs=(
            pl.BlockSpec((1, H, LB, HD), lambda b, i: (b, 0, i, 0)),
            row_spec,
            row_spec,
            pl.BlockSpec((1, LB, H * P), lambda b, i: (b, i, 0)),
            pl.BlockSpec((1, LB, H * P), lambda b, i: (b, i, 0)),
        ),
        out_shape=out_shapes,
    )(q_in, kv_in, Wq, bq, Wk, bk, Wv, bv, Woff, boff, Wattn, battn)


def _build_body(k_ref, kn_ref, v_ref, vn_ref, kvp_ref):
    hd = kvp_ref.shape[-1] // 4
    k = k_ref[0]
    kn = kn_ref[0]
    v = v_ref[0]
    vn = vn_ref[0]
    for h in range(H):
        sl = slice(h * hd, (h + 1) * hd)
        kh = k[:, sl]
        vh = v[:, sl]
        kh1 = jnp.concatenate([kh[1:], kn[:1, sl]], axis=0)
        vh1 = jnp.concatenate([vh[1:], vn[:1, sl]], axis=0)
        kvp_ref[0, h] = jnp.concatenate([kh, kh1, vh, vh1], axis=1)


def _build_pairs(k, v):
    B, L, D = k.shape
    HD = D // H
    LB = 256
    nblk = L // LB
    grid = (B, nblk)
    cur = pl.BlockSpec((1, LB, D), lambda b, i: (b, i, 0))
    nxt = pl.BlockSpec((1, LB, D),
                       lambda b, i: (b, jnp.minimum(i + 1, nblk - 1), 0))
    return pl.pallas_call(
        _build_body,
        grid=grid,
        in_specs=[cur, nxt, cur, nxt],
        out_specs=pl.BlockSpec((1, H, LB, 4 * HD), lambda b, i: (b, 0, i, 0)),
        out_shape=jax.ShapeDtypeStruct((B, H, L, 4 * HD), jnp.float32),
    )(k, k, v, v)


def _mm(a, b):
    return jnp.dot(a, b, preferred_element_type=jnp.float32)


def _attn_body(q_ref, g_ref, meta_ref, ctx_ref):
    hd = q_ref.shape[-1]
    q = q_ref[0, 0]
    meta = meta_ref[0, 0]
    segcol = jax.lax.broadcasted_iota(jnp.int32, (2 * P, 2 * P * hd), 1)
    segrow = jax.lax.broadcasted_iota(jnp.int32, (2 * P, 2 * P * hd), 0)
    src = (segcol // hd) % 2 * P + segcol // (2 * hd)
    WC = (segrow == src).astype(jnp.float32)
    drow = jax.lax.broadcasted_iota(jnp.int32, (hd, 2 * P * hd), 0)
    dcol = jax.lax.broadcasted_iota(jnp.int32, (hd, 2 * P * hd), 1)
    QE = (drow == dcol % hd).astype(jnp.float32)
    prow = jax.lax.broadcasted_iota(jnp.int32, (2 * P * hd, P), 0)
    pcol = jax.lax.broadcasted_iota(jnp.int32, (2 * P * hd, P), 1)
    S4 = (prow // (2 * hd) == pcol).astype(jnp.float32)

    kcat = jnp.concatenate([g_ref[p, 0, 0][:, :2 * hd] for p in range(P)],
                           axis=1)
    wbig = _mm(meta[:, :2 * P], WC)
    qbig = _mm(q, QE)
    prod = kcat * wbig * qbig
    scores = jnp.dot(prod.astype(jnp.bfloat16), S4.astype(jnp.bfloat16),
                     preferred_element_type=jnp.float32)
    scores = scores + meta[:, 2 * P:3 * P]
    m = jnp.max(scores, axis=1, keepdims=True)
    e = jnp.exp(scores - m)
    z = jnp.sum(e, axis=1, keepdims=True)
    wgt = e / z

    vcat = jnp.concatenate([g_ref[p, 0, 0][:, 2 * hd:] for p in range(P)],
                           axis=1)
    ccat = jnp.concatenate([wgt * meta[:, :P], wgt * meta[:, P:2 * P]],
                           axis=1)
    cbig = _mm(ccat, WC)
    ctxfull = vcat * cbig
    ctx = ctxfull[:, :hd]
    for s in range(1, 2 * P):
        ctx = ctx + ctxfull[:, s * hd:(s + 1) * hd]
    ctx_ref[0, 0] = ctx


def _attention(qh, gath5, meta, h0, hg):
    B, Hh, L, HD = qh.shape
    LB = 1024
    grid = (B, hg, L // LB)
    return pl.pallas_call(
        _attn_body,
        grid=grid,
        in_specs=[
            pl.BlockSpec((1, 1, LB, HD), lambda b, h, i: (b, h0 + h, i, 0)),
            pl.BlockSpec((P, 1, 1, LB, 4 * HD), lambda b, h, i: (0, b, h, i, 0)),
            pl.BlockSpec((1, 1, LB, 64), lambda b, h, i: (b, h0 + h, i, 0)),
        ],
        out_specs=pl.BlockSpec((1, 1, LB, HD), lambda b, h, i: (b, h, i, 0)),
        out_shape=jax.ShapeDtypeStruct((B, hg, L, HD), jnp.float32),
    )(qh, gath5, meta)


def _outproj_body(c0_ref, c1_ref, c2_ref, c3_ref, w_ref, b_ref, o_ref):
    hg = c0_ref.shape[1]
    x = jnp.concatenate(
        [c_ref[0, h] for c_ref in (c0_ref, c1_ref, c2_ref, c3_ref)
         for h in range(hg)], axis=1)
    o_ref[0] = jnp.dot(x, w_ref[...], preferred_element_type=jnp.float32) + b_ref[...]


def _outproj(ctxs, Wout, bout):
    B, hg, L, HD = ctxs[0].shape
    D = H * HD
    LB = 256
    grid = (B, L // LB)
    cspec = pl.BlockSpec((1, hg, LB, HD), lambda b, i: (b, 0, i, 0))
    return pl.pallas_call(
        _outproj_body,
        grid=grid,
        in_specs=[
            cspec, cspec, cspec, cspec,
            pl.BlockSpec((D, D), lambda b, i: (0, 0)),
            pl.BlockSpec((D,), lambda b, i: (0,)),
        ],
        out_specs=pl.BlockSpec((1, LB, D), lambda b, i: (b, i, 0)),
        out_shape=jax.ShapeDtypeStruct((B, L, D), jnp.float32),
    )(*ctxs, Wout, bout)


def _sc_gather(table, gidx):
    R = gidx.shape[0]
    W = table.shape[1]
    NC = 2
    NS = 16
    NW = NC * NS
    r_per_w = R // NW
    CH = 64
    NBUF = 4
    n_chunks = r_per_w // CH
    mesh = plsc.VectorSubcoreMesh(core_axis_name="c", subcore_axis_name="s")

    @functools.partial(
        pl.kernel,
        out_type=jax.ShapeDtypeStruct((R, W), jnp.float32),
        mesh=mesh,
        scratch_types=[
            pltpu.VMEM((r_per_w,), jnp.int32),
            pltpu.VMEM((NBUF, CH, W), jnp.float32),
            pltpu.SemaphoreType.DMA((NBUF,)),
            pltpu.SemaphoreType.DMA((NBUF,)),
        ],
    )
    def gather_kernel(tab_hbm, idx_hbm, out_hbm, idx_v, rows_v, sem_g, sem_o):
        wid = lax.axis_index("s") * NC + lax.axis_index("c")
        base = wid * r_per_w
        pltpu.sync_copy(idx_hbm.at[pl.ds(base, r_per_w)], idx_v)

        def fill(c, b):
            pltpu.async_copy(tab_hbm.at[idx_v.at[pl.ds(c * CH, CH)]],
                             rows_v.at[b], sem_g.at[b])

        def wait_fill(c, b):
            pltpu.make_async_copy(tab_hbm.at[idx_v.at[pl.ds(c * CH, CH)]],
                                  rows_v.at[b], sem_g.at[b]).wait()

        def drain(c, b):
            pltpu.async_copy(rows_v.at[b], out_hbm.at[pl.ds(base + c * CH, CH)],
                             sem_o.at[b])

        def wait_drain(c, b):
            pltpu.make_async_copy(rows_v.at[b],
                                  out_hbm.at[pl.ds(base + c * CH, CH)],
                                  sem_o.at[b]).wait()

        for b in range(NBUF):
            fill(b, b)

        @pl.loop(0, n_chunks - NBUF, step=NBUF)
        def _(c):
            for b in range(NBUF):
                wait_fill(c + b, b)
                drain(c + b, b)
            for b in range(NBUF):
                wait_drain(c + b, b)
                fill(c + NBUF + b, b)

        for b in range(NBUF):
            wait_fill(n_chunks - NBUF + b, b)
            drain(n_chunks - NBUF + b, b)
        for b in range(NBUF):
            wait_drain(n_chunks - NBUF + b, b)

    return gather_kernel(table, gidx)


@jax.jit
def kernel(q_in, kv_in, Wq, bq, Wk, bk, Wv, bv, Woff, boff, Wattn, battn,
           Wout, bout):
    B, L, D = q_in.shape
    HD = D // H

    qh, k, v, off, lg = _projections(q_in, kv_in, Wq, bq, Wk, bk, Wv, bv,
                                     Woff, boff, Wattn, battn)
    kvp = _build_pairs(k, v)

    offT = off.reshape(B, L, H, P).transpose(0, 2, 1, 3)
    lgT = lg.reshape(B, L, H, P).transpose(0, 2, 1, 3)
    basef = jnp.arange(L, dtype=jnp.float32).reshape(1, 1, L, 1)
    idxf = jnp.clip(basef + offT, 0.0, float(L - 1))
    base = jnp.clip(jnp.floor(idxf), 0.0, float(L - 2))
    w1 = idxf - base
    w0 = 1.0 - w1
    meta = jnp.concatenate(
        [w0, w1, lgT, jnp.zeros((B, H, L, 64 - 3 * P), jnp.float32)], axis=-1)
    bh = jnp.arange(B * H, dtype=jnp.int32).reshape(B, H, 1, 1)
    rowid = (bh * L + base.astype(jnp.int32)).transpose(3, 0, 1, 2)

    G = 4
    hg = H // G
    kvp_flat = kvp.reshape(B * H * L, 4 * HD)
    gaths = [
        _sc_gather(kvp_flat, rowid[:, :, g * hg:(g + 1) * hg, :].reshape(-1))
        for g in range(G)
    ]
    ctxs = [
        _attention(qh, gaths[g].reshape(P, B, hg, L, 4 * HD), meta,
                   g * hg, hg)
        for g in range(G)
    ]
    return _outproj(ctxs, Wout, bout)


# speedup vs baseline: 1.1486x; 1.0065x over previous
import functools
import math

import jax
import jax.numpy as jnp
from jax import lax
from jax.experimental import pallas as pl
from jax.experimental.pallas import tpu as pltpu
from jax.experimental.pallas import tpu_sc as plsc

H = 16
P = 4


def _proj_body(x_ref, kv_ref, wq_ref, bq_ref, wk_ref, bk_ref, wv_ref, bv_ref,
               woff_ref, boff_ref, wattn_ref, battn_ref,
               qh_ref, k_ref, v_ref, off_ref, lg_ref):
    hd = qh_ref.shape[-1]
    x = x_ref[0]
    kv = kv_ref[0]
    q = jnp.dot(x, wq_ref[...], preferred_element_type=jnp.float32) + bq_ref[...]
    k = jnp.dot(kv, wk_ref[...], preferred_element_type=jnp.float32) + bk_ref[...]
    v = jnp.dot(kv, wv_ref[...], preferred_element_type=jnp.float32) + bv_ref[...]
    off = jnp.dot(x, woff_ref[...], preferred_element_type=jnp.float32) + boff_ref[...]
    lg = jnp.dot(x, wattn_ref[...], preferred_element_type=jnp.float32) + battn_ref[...]
    k_ref[0] = k
    v_ref[0] = v
    off_ref[0] = off
    lg_ref[0] = lg
    scale = 1.0 / math.sqrt(hd)
    for h in range(H):
        qh_ref[0, h] = q[:, h * hd:(h + 1) * hd] * scale


def _projections(q_in, kv_in, Wq, bq, Wk, bk, Wv, bv, Woff, boff, Wattn, battn):
    B, L, D = q_in.shape
    HD = D // H
    LB = 256
    grid = (B, L // LB)
    full = lambda shape: pl.BlockSpec(shape, lambda b, i: (0,) * len(shape))
    row_spec = pl.BlockSpec((1, LB, D), lambda b, i: (b, i, 0))
    out_shapes = (
        jax.ShapeDtypeStruct((B, H, L, HD), jnp.float32),
        jax.ShapeDtypeStruct((B, L, D), jnp.float32),
        jax.ShapeDtypeStruct((B, L, D), jnp.float32),
        jax.ShapeDtypeStruct((B, L, H * P), jnp.float32),
        jax.ShapeDtypeStruct((B, L, H * P), jnp.float32),
    )
    return pl.pallas_call(
        _proj_body,
        grid=grid,
        in_specs=[
            row_spec, row_spec,
            full((D, D)), full((D,)), full((D, D)), full((D,)),
            full((D, D)), full((D,)), full((D, H * P)), full((H * P,)),
            full((D, H * P)), full((H * P,)),
        ],
        out_specs=(
            pl.BlockSpec((1, H, LB, HD), lambda b, i: (b, 0, i, 0)),
            row_spec,
            row_spec,
            pl.BlockSpec((1, LB, H * P), lambda b, i: (b, i, 0)),
            pl.BlockSpec((1, LB, H * P), lambda b, i: (b, i, 0)),
        ),
        out_shape=out_shapes,
    )(q_in, kv_in, Wq, bq, Wk, bk, Wv, bv, Woff, boff, Wattn, battn)


def _build_body(k_ref, kn_ref, v_ref, vn_ref, kvp_ref):
    hd = kvp_ref.shape[-1] // 4
    k = k_ref[0]
    kn = kn_ref[0]
    v = v_ref[0]
    vn = vn_ref[0]
    for h in range(H):
        sl = slice(h * hd, (h + 1) * hd)
        kh = k[:, sl]
        vh = v[:, sl]
        kh1 = jnp.concatenate([kh[1:], kn[:1, sl]], axis=0)
        vh1 = jnp.concatenate([vh[1:], vn[:1, sl]], axis=0)
        kvp_ref[0, h] = jnp.concatenate([kh, kh1, vh, vh1], axis=1)


def _build_pairs(k, v):
    B, L, D = k.shape
    HD = D // H
    LB = 256
    nblk = L // LB
    grid = (B, nblk)
    cur = pl.BlockSpec((1, LB, D), lambda b, i: (b, i, 0))
    nxt = pl.BlockSpec((1, LB, D),
                       lambda b, i: (b, jnp.minimum(i + 1, nblk - 1), 0))
    return pl.pallas_call(
        _build_body,
        grid=grid,
        in_specs=[cur, nxt, cur, nxt],
        out_specs=pl.BlockSpec((1, H, LB, 4 * HD), lambda b, i: (b, 0, i, 0)),
        out_shape=jax.ShapeDtypeStruct((B, H, L, 4 * HD), jnp.float32),
    )(k, k, v, v)


def _mm(a, b):
    return jnp.dot(a, b, preferred_element_type=jnp.float32)


def _attn_body(q_ref, g_ref, meta_ref, ctx_ref):
    hd = q_ref.shape[-1]
    q = q_ref[0, 0]
    meta = meta_ref[0, 0]
    segcol = jax.lax.broadcasted_iota(jnp.int32, (2 * P, 2 * P * hd), 1)
    segrow = jax.lax.broadcasted_iota(jnp.int32, (2 * P, 2 * P * hd), 0)
    src = (segcol // hd) % 2 * P + segcol // (2 * hd)
    WC = (segrow == src).astype(jnp.float32)
    drow = jax.lax.broadcasted_iota(jnp.int32, (hd, 2 * P * hd), 0)
    dcol = jax.lax.broadcasted_iota(jnp.int32, (hd, 2 * P * hd), 1)
    QE = (drow == dcol % hd).astype(jnp.float32)
    prow = jax.lax.broadcasted_iota(jnp.int32, (2 * P * hd, P), 0)
    pcol = jax.lax.broadcasted_iota(jnp.int32, (2 * P * hd, P), 1)
    S4 = (prow // (2 * hd) == pcol).astype(jnp.float32)

    kcat = jnp.concatenate([g_ref[p, 0, 0][:, :2 * hd] for p in range(P)],
                           axis=1)
    wbig = _mm(meta[:, :2 * P], WC)
    qbig = _mm(q, QE)
    prod = kcat * wbig * qbig
    scores = jnp.dot(prod.astype(jnp.bfloat16), S4.astype(jnp.bfloat16),
                     preferred_element_type=jnp.float32)
    scores = scores + meta[:, 2 * P:3 * P]
    m = jnp.max(scores, axis=1, keepdims=True)
    e = jnp.exp(scores - m)
    z = jnp.sum(e, axis=1, keepdims=True)
    wgt = e / z

    vcat = jnp.concatenate([g_ref[p, 0, 0][:, 2 * hd:] for p in range(P)],
                           axis=1)
    ccat = jnp.concatenate([wgt * meta[:, :P], wgt * meta[:, P:2 * P]],
                           axis=1)
    cbig = _mm(ccat, WC)
    ctxfull = vcat * cbig
    ctx = ctxfull[:, :hd]
    for s in range(1, 2 * P):
        ctx = ctx + ctxfull[:, s * hd:(s + 1) * hd]
    ctx_ref[0, 0] = ctx


def _attention(qh, gath5, meta, h0, hg):
    B, Hh, L, HD = qh.shape
    LB = 1024
    grid = (B, hg, L // LB)
    return pl.pallas_call(
        _attn_body,
        grid=grid,
        in_specs=[
            pl.BlockSpec((1, 1, LB, HD), lambda b, h, i: (b, h0 + h, i, 0)),
            pl.BlockSpec((P, 1, 1, LB, 4 * HD), lambda b, h, i: (0, b, h, i, 0)),
            pl.BlockSpec((1, 1, LB, 64), lambda b, h, i: (b, h0 + h, i, 0)),
        ],
        out_specs=pl.BlockSpec((1, 1, LB, HD), lambda b, h, i: (b, h, i, 0)),
        out_shape=jax.ShapeDtypeStruct((B, hg, L, HD), jnp.float32),
    )(qh, gath5, meta)


def _outproj_body(*refs):
    c_refs = refs[:-3]
    w_ref, b_ref, o_ref = refs[-3:]
    hg = c_refs[0].shape[1]
    x = jnp.concatenate(
        [c_ref[0, h] for c_ref in c_refs for h in range(hg)], axis=1)
    o_ref[0] = jnp.dot(x, w_ref[...], preferred_element_type=jnp.float32) + b_ref[...]


def _outproj(ctxs, Wout, bout):
    B, hg, L, HD = ctxs[0].shape
    D = H * HD
    LB = 256
    grid = (B, L // LB)
    cspec = pl.BlockSpec((1, hg, LB, HD), lambda b, i: (b, 0, i, 0))
    return pl.pallas_call(
        _outproj_body,
        grid=grid,
        in_specs=[cspec] * len(ctxs) + [
            pl.BlockSpec((D, D), lambda b, i: (0, 0)),
            pl.BlockSpec((D,), lambda b, i: (0,)),
        ],
        out_specs=pl.BlockSpec((1, LB, D), lambda b, i: (b, i, 0)),
        out_shape=jax.ShapeDtypeStruct((B, L, D), jnp.float32),
    )(*ctxs, Wout, bout)


def _sc_gather(table, gidx):
    R = gidx.shape[0]
    W = table.shape[1]
    NC = 2
    NS = 16
    NW = NC * NS
    r_per_w = R // NW
    CH = 64
    NBUF = 4
    n_chunks = r_per_w // CH
    mesh = plsc.VectorSubcoreMesh(core_axis_name="c", subcore_axis_name="s")

    @functools.partial(
        pl.kernel,
        out_type=jax.ShapeDtypeStruct((R, W), jnp.float32),
        mesh=mesh,
        scratch_types=[
            pltpu.VMEM((r_per_w,), jnp.int32),
            pltpu.VMEM((NBUF, CH, W), jnp.float32),
            pltpu.SemaphoreType.DMA((NBUF,)),
            pltpu.SemaphoreType.DMA((NBUF,)),
        ],
    )
    def gather_kernel(tab_hbm, idx_hbm, out_hbm, idx_v, rows_v, sem_g, sem_o):
        wid = lax.axis_index("s") * NC + lax.axis_index("c")
        base = wid * r_per_w
        pltpu.sync_copy(idx_hbm.at[pl.ds(base, r_per_w)], idx_v)

        def fill(c, b):
            pltpu.async_copy(tab_hbm.at[idx_v.at[pl.ds(c * CH, CH)]],
                             rows_v.at[b], sem_g.at[b])

        def wait_fill(c, b):
            pltpu.make_async_copy(tab_hbm.at[idx_v.at[pl.ds(c * CH, CH)]],
                                  rows_v.at[b], sem_g.at[b]).wait()

        def drain(c, b):
            pltpu.async_copy(rows_v.at[b], out_hbm.at[pl.ds(base + c * CH, CH)],
                             sem_o.at[b])

        def wait_drain(c, b):
            pltpu.make_async_copy(rows_v.at[b],
                                  out_hbm.at[pl.ds(base + c * CH, CH)],
                                  sem_o.at[b]).wait()

        for b in range(NBUF):
            fill(b, b)

        @pl.loop(0, n_chunks - NBUF, step=NBUF)
        def _(c):
            for b in range(NBUF):
                wait_fill(c + b, b)
                drain(c + b, b)
            for b in range(NBUF):
                wait_drain(c + b, b)
                fill(c + NBUF + b, b)

        for b in range(NBUF):
            wait_fill(n_chunks - NBUF + b, b)
            drain(n_chunks - NBUF + b, b)
        for b in range(NBUF):
            wait_drain(n_chunks - NBUF + b, b)

    return gather_kernel(table, gidx)


@jax.jit
def kernel(q_in, kv_in, Wq, bq, Wk, bk, Wv, bv, Woff, boff, Wattn, battn,
           Wout, bout):
    B, L, D = q_in.shape
    HD = D // H

    qh, k, v, off, lg = _projections(q_in, kv_in, Wq, bq, Wk, bk, Wv, bv,
                                     Woff, boff, Wattn, battn)
    kvp = _build_pairs(k, v)

    offT = off.reshape(B, L, H, P).transpose(0, 2, 1, 3)
    lgT = lg.reshape(B, L, H, P).transpose(0, 2, 1, 3)
    basef = jnp.arange(L, dtype=jnp.float32).reshape(1, 1, L, 1)
    idxf = jnp.clip(basef + offT, 0.0, float(L - 1))
    base = jnp.clip(jnp.floor(idxf), 0.0, float(L - 2))
    w1 = idxf - base
    w0 = 1.0 - w1
    meta = jnp.concatenate(
        [w0, w1, lgT, jnp.zeros((B, H, L, 64 - 3 * P), jnp.float32)], axis=-1)
    bh = jnp.arange(B * H, dtype=jnp.int32).reshape(B, H, 1, 1)
    rowid = (bh * L + base.astype(jnp.int32)).transpose(3, 0, 1, 2)

    G = 2
    hg = H // G
    kvp_flat = kvp.reshape(B * H * L, 4 * HD)
    gaths = [
        _sc_gather(kvp_flat, rowid[:, :, g * hg:(g + 1) * hg, :].reshape(-1))
        for g in range(G)
    ]
    ctxs = [
        _attention(qh, gaths[g].reshape(P, B, hg, L, 4 * HD), meta,
                   g * hg, hg)
        for g in range(G)
    ]
    return _outproj(ctxs, Wout, bout)


# speedup vs baseline: 1.1520x; 1.0030x over previous
import functools
import math

import jax
import jax.numpy as jnp
from jax import lax
from jax.experimental import pallas as pl
from jax.experimental.pallas import tpu as pltpu
from jax.experimental.pallas import tpu_sc as plsc

H = 16
P = 4


def _proj_body(x_ref, kv_ref, wq_ref, bq_ref, wk_ref, bk_ref, wv_ref, bv_ref,
               woff_ref, boff_ref, wattn_ref, battn_ref,
               qh_ref, k_ref, v_ref, off_ref, lg_ref):
    hd = qh_ref.shape[-1]
    x = x_ref[0]
    kv = kv_ref[0]
    q = jnp.dot(x, wq_ref[...], preferred_element_type=jnp.float32) + bq_ref[...]
    k = jnp.dot(kv, wk_ref[...], preferred_element_type=jnp.float32) + bk_ref[...]
    v = jnp.dot(kv, wv_ref[...], preferred_element_type=jnp.float32) + bv_ref[...]
    off = jnp.dot(x, woff_ref[...], preferred_element_type=jnp.float32) + boff_ref[...]
    lg = jnp.dot(x, wattn_ref[...], preferred_element_type=jnp.float32) + battn_ref[...]
    k_ref[0] = k
    v_ref[0] = v
    off_ref[0] = off
    lg_ref[0] = lg
    scale = 1.0 / math.sqrt(hd)
    for h in range(H):
        qh_ref[0, h] = q[:, h * hd:(h + 1) * hd] * scale


def _projections(q_in, kv_in, Wq, bq, Wk, bk, Wv, bv, Woff, boff, Wattn, battn):
    B, L, D = q_in.shape
    HD = D // H
    LB = 256
    grid = (B, L // LB)
    full = lambda shape: pl.BlockSpec(shape, lambda b, i: (0,) * len(shape))
    row_spec = pl.BlockSpec((1, LB, D), lambda b, i: (b, i, 0))
    out_shapes = (
        jax.ShapeDtypeStruct((B, H, L, HD), jnp.float32),
        jax.ShapeDtypeStruct((B, L, D), jnp.float32),
        jax.ShapeDtypeStruct((B, L, D), jnp.float32),
        jax.ShapeDtypeStruct((B, L, H * P), jnp.float32),
        jax.ShapeDtypeStruct((B, L, H * P), jnp.float32),
    )
    return pl.pallas_call(
        _proj_body,
        grid=grid,
        in_specs=[
            row_spec, row_spec,
            full((D, D)), full((D,)), full((D, D)), full((D,)),
            full((D, D)), full((D,)), full((D, H * P)), full((H * P,)),
            full((D, H * P)), full((H * P,)),
        ],
        out_specs=(
            pl.BlockSpec((1, H, LB, HD), lambda b, i: (b, 0, i, 0)),
            row_spec,
            row_spec,
            pl.BlockSpec((1, LB, H * P), lambda b, i: (b, i, 0)),
            pl.BlockSpec((1, LB, H * P), lambda b, i: (b, i, 0)),
        ),
        out_shape=out_shapes,
    )(q_in, kv_in, Wq, bq, Wk, bk, Wv, bv, Woff, boff, Wattn, battn)


def _build_body(k_ref, kn_ref, v_ref, vn_ref, kvp_ref):
    hd = kvp_ref.shape[-1] // 4
    k = k_ref[0]
    kn = kn_ref[0]
    v = v_ref[0]
    vn = vn_ref[0]
    for h in range(H):
        sl = slice(h * hd, (h + 1) * hd)
        kh = k[:, sl]
        vh = v[:, sl]
        kh1 = jnp.concatenate([kh[1:], kn[:1, sl]], axis=0)
        vh1 = jnp.concatenate([vh[1:], vn[:1, sl]], axis=0)
        kvp_ref[0, h] = jnp.concatenate([kh, kh1, vh, vh1], axis=1)


def _build_pairs(k, v):
    B, L, D = k.shape
    HD = D // H
    LB = 256
    nblk = L // LB
    grid = (B, nblk)
    cur = pl.BlockSpec((1, LB, D), lambda b, i: (b, i, 0))
    nxt = pl.BlockSpec((1, LB, D),
                       lambda b, i: (b, jnp.minimum(i + 1, nblk - 1), 0))
    return pl.pallas_call(
        _build_body,
        grid=grid,
        in_specs=[cur, nxt, cur, nxt],
        out_specs=pl.BlockSpec((1, H, LB, 4 * HD), lambda b, i: (b, 0, i, 0)),
        out_shape=jax.ShapeDtypeStruct((B, H, L, 4 * HD), jnp.float32),
    )(k, k, v, v)


def _mm(a, b):
    return jnp.dot(a, b, preferred_element_type=jnp.float32)


def _attn_body(q_ref, g_ref, meta_ref, ctx_ref):
    hd = q_ref.shape[-1]
    q = q_ref[0, 0]
    meta = meta_ref[0, 0]
    segcol = jax.lax.broadcasted_iota(jnp.int32, (2 * P, 2 * P * hd), 1)
    segrow = jax.lax.broadcasted_iota(jnp.int32, (2 * P, 2 * P * hd), 0)
    src = (segcol // hd) % 2 * P + segcol // (2 * hd)
    WC = (segrow == src).astype(jnp.float32)
    drow = jax.lax.broadcasted_iota(jnp.int32, (hd, 2 * P * hd), 0)
    dcol = jax.lax.broadcasted_iota(jnp.int32, (hd, 2 * P * hd), 1)
    QE = (drow == dcol % hd).astype(jnp.float32)
    prow = jax.lax.broadcasted_iota(jnp.int32, (2 * P * hd, P), 0)
    pcol = jax.lax.broadcasted_iota(jnp.int32, (2 * P * hd, P), 1)
    S4 = (prow // (2 * hd) == pcol).astype(jnp.float32)

    kcat = jnp.concatenate([g_ref[p, 0, 0][:, :2 * hd] for p in range(P)],
                           axis=1)
    wbig = _mm(meta[:, :2 * P], WC)
    qbig = _mm(q, QE)
    prod = kcat * wbig * qbig
    scores = jnp.dot(prod.astype(jnp.bfloat16), S4.astype(jnp.bfloat16),
                     preferred_element_type=jnp.float32)
    scores = scores + meta[:, 2 * P:3 * P]
    m = jnp.max(scores, axis=1, keepdims=True)
    e = jnp.exp(scores - m)
    z = jnp.sum(e, axis=1, keepdims=True)
    wgt = e / z

    vcat = jnp.concatenate([g_ref[p, 0, 0][:, 2 * hd:] for p in range(P)],
                           axis=1)
    ccat = jnp.concatenate([wgt * meta[:, :P], wgt * meta[:, P:2 * P]],
                           axis=1)
    cbig = _mm(ccat, WC)
    ctxfull = vcat * cbig
    ctx = ctxfull[:, :hd]
    for s in range(1, 2 * P):
        ctx = ctx + ctxfull[:, s * hd:(s + 1) * hd]
    ctx_ref[0, 0] = ctx


def _attention(qh, gath5, meta, h0, hg):
    B, Hh, L, HD = qh.shape
    LB = 1024
    grid = (B, hg, L // LB)
    return pl.pallas_call(
        _attn_body,
        grid=grid,
        in_specs=[
            pl.BlockSpec((1, 1, LB, HD), lambda b, h, i: (b, h0 + h, i, 0)),
            pl.BlockSpec((P, 1, 1, LB, 4 * HD), lambda b, h, i: (0, b, h, i, 0)),
            pl.BlockSpec((1, 1, LB, 64), lambda b, h, i: (b, h0 + h, i, 0)),
        ],
        out_specs=pl.BlockSpec((1, 1, LB, HD), lambda b, h, i: (b, h, i, 0)),
        out_shape=jax.ShapeDtypeStruct((B, hg, L, HD), jnp.float32),
    )(qh, gath5, meta)


def _outproj_body(*refs):
    c_refs = refs[:-3]
    w_ref, b_ref, o_ref = refs[-3:]
    hg = c_refs[0].shape[1]
    x = jnp.concatenate(
        [c_ref[0, h] for c_ref in c_refs for h in range(hg)], axis=1)
    o_ref[0] = jnp.dot(x, w_ref[...], preferred_element_type=jnp.float32) + b_ref[...]


def _outproj(ctxs, Wout, bout):
    B, hg, L, HD = ctxs[0].shape
    D = H * HD
    LB = 256
    grid = (B, L // LB)
    cspec = pl.BlockSpec((1, hg, LB, HD), lambda b, i: (b, 0, i, 0))
    return pl.pallas_call(
        _outproj_body,
        grid=grid,
        in_specs=[cspec] * len(ctxs) + [
            pl.BlockSpec((D, D), lambda b, i: (0, 0)),
            pl.BlockSpec((D,), lambda b, i: (0,)),
        ],
        out_specs=pl.BlockSpec((1, LB, D), lambda b, i: (b, i, 0)),
        out_shape=jax.ShapeDtypeStruct((B, L, D), jnp.float32),
    )(*ctxs, Wout, bout)


def _sc_gather(table, gidx):
    R = gidx.shape[0]
    W = table.shape[1]
    NC = 2
    NS = 16
    NW = NC * NS
    r_per_w = R // NW
    CH = 128
    NBUF = 2
    assert (R // NW) % CH == 0 and (R // NW // CH) % NBUF == 0
    n_chunks = r_per_w // CH
    mesh = plsc.VectorSubcoreMesh(core_axis_name="c", subcore_axis_name="s")

    @functools.partial(
        pl.kernel,
        out_type=jax.ShapeDtypeStruct((R, W), jnp.float32),
        mesh=mesh,
        scratch_types=[
            pltpu.VMEM((r_per_w,), jnp.int32),
            pltpu.VMEM((NBUF, CH, W), jnp.float32),
            pltpu.SemaphoreType.DMA((NBUF,)),
            pltpu.SemaphoreType.DMA((NBUF,)),
        ],
    )
    def gather_kernel(tab_hbm, idx_hbm, out_hbm, idx_v, rows_v, sem_g, sem_o):
        wid = lax.axis_index("s") * NC + lax.axis_index("c")
        base = wid * r_per_w
        pltpu.sync_copy(idx_hbm.at[pl.ds(base, r_per_w)], idx_v)

        def fill(c, b):
            pltpu.async_copy(tab_hbm.at[idx_v.at[pl.ds(c * CH, CH)]],
                             rows_v.at[b], sem_g.at[b])

        def wait_fill(c, b):
            pltpu.make_async_copy(tab_hbm.at[idx_v.at[pl.ds(c * CH, CH)]],
                                  rows_v.at[b], sem_g.at[b]).wait()

        def drain(c, b):
            pltpu.async_copy(rows_v.at[b], out_hbm.at[pl.ds(base + c * CH, CH)],
                             sem_o.at[b])

        def wait_drain(c, b):
            pltpu.make_async_copy(rows_v.at[b],
                                  out_hbm.at[pl.ds(base + c * CH, CH)],
                                  sem_o.at[b]).wait()

        for b in range(NBUF):
            fill(b, b)

        @pl.loop(0, n_chunks - NBUF, step=NBUF)
        def _(c):
            for b in range(NBUF):
                wait_fill(c + b, b)
                drain(c + b, b)
            for b in range(NBUF):
                wait_drain(c + b, b)
                fill(c + NBUF + b, b)

        for b in range(NBUF):
            wait_fill(n_chunks - NBUF + b, b)
            drain(n_chunks - NBUF + b, b)
        for b in range(NBUF):
            wait_drain(n_chunks - NBUF + b, b)

    return gather_kernel(table, gidx)


@jax.jit
def kernel(q_in, kv_in, Wq, bq, Wk, bk, Wv, bv, Woff, boff, Wattn, battn,
           Wout, bout):
    B, L, D = q_in.shape
    HD = D // H

    qh, k, v, off, lg = _projections(q_in, kv_in, Wq, bq, Wk, bk, Wv, bv,
                                     Woff, boff, Wattn, battn)
    kvp = _build_pairs(k, v)

    offT = off.reshape(B, L, H, P).transpose(0, 2, 1, 3)
    lgT = lg.reshape(B, L, H, P).transpose(0, 2, 1, 3)
    basef = jnp.arange(L, dtype=jnp.float32).reshape(1, 1, L, 1)
    idxf = jnp.clip(basef + offT, 0.0, float(L - 1))
    base = jnp.clip(jnp.floor(idxf), 0.0, float(L - 2))
    w1 = idxf - base
    w0 = 1.0 - w1
    meta = jnp.concatenate(
        [w0, w1, lgT, jnp.zeros((B, H, L, 64 - 3 * P), jnp.float32)], axis=-1)
    bh = jnp.arange(B * H, dtype=jnp.int32).reshape(B, H, 1, 1)
    rowid = (bh * L + base.astype(jnp.int32)).transpose(3, 0, 1, 2)

    G = 2
    hg = H // G
    kvp_flat = kvp.reshape(B * H * L, 4 * HD)
    gaths = [
        _sc_gather(kvp_flat, rowid[:, :, g * hg:(g + 1) * hg, :].reshape(-1))
        for g in range(G)
    ]
    ctxs = [
        _attention(qh, gaths[g].reshape(P, B, hg, L, 4 * HD), meta,
                   g * hg, hg)
        for g in range(G)
    ]
    return _outproj(ctxs, Wout, bout)
